# Initial kernel scaffold; baseline (speedup 1.0000x reference)
import jax
import jax.numpy as jnp
from jax.experimental import pallas as pl


def kernel(x, edge_index, batch, Wq1, bq1, Wk1, bk1, Wv1, bv1, Ws1, bs1, Wq2, bq2, Wk2, bk2, Wv2, bv2, Ws2, bs2, Wfc1, bfc1, Wfc2, bfc2):
    raise NotImplementedError("write your pallas kernel here")


# speedup vs baseline: 36.5036x; 36.5036x over previous
import functools

import jax
import jax.numpy as jnp
from jax import lax
from jax.experimental import pallas as pl
from jax.experimental.pallas import tpu as pltpu
from jax.experimental.pallas import tpu_sc as plsc

NC, NS = 2, 16
NW = NC * NS
NUM_G = 512
BN = 2048
CH = 256
GC = CH // 128
ZR = 128
NP = 102400
F32 = jnp.float32


def _tc_qkv(x_p, Wq, bq, Wk, bk, Wv, bv, scale):
    n, f = x_p.shape
    d = Wq.shape[1]

    def body(x_ref, wq, bq_r, wk, bk_r, wv, bv_r, q_o, k_o, v_o):
        xb = x_ref[...]
        q_o[...] = (jnp.dot(xb, wq[...], preferred_element_type=F32)
                    + bq_r[...]) * scale
        k_o[...] = jnp.dot(xb, wk[...], preferred_element_type=F32) + bk_r[...]
        v_o[...] = jnp.dot(xb, wv[...], preferred_element_type=F32) + bv_r[...]

    full = lambda a: pl.BlockSpec(a.shape, lambda i: (0,) * a.ndim)
    return pl.pallas_call(
        body,
        grid=(n // BN,),
        in_specs=[pl.BlockSpec((BN, f), lambda i: (i, 0)),
                  full(Wq), full(bq), full(Wk), full(bk), full(Wv), full(bv)],
        out_specs=[pl.BlockSpec((BN, d), lambda i: (i, 0))] * 3,
        out_shape=[jax.ShapeDtypeStruct((n, d), F32)] * 3,
    )(x_p, Wq, bq, Wk, bk, Wv, bv)


def _tc_finalize1_qkv2(num1, s1, x_p, Ws1, bs1, Wq2, bq2, Wk2, bk2,
                       Wvlo, bvlo, Wvhi, bvhi, scale2):
    n, f = x_p.shape

    def body(num_ref, s_ref, x_ref, ws1, bs1_r, wq2, bq2_r, wk2, bk2_r,
             wvlo, bvlo_r, wvhi, bvhi_r, h1_o, q2_o, k2_o, vlo_o, vhi_o):
        num = num_ref[0] + num_ref[1]
        s = s_ref[0] + s_ref[1]
        xb = x_ref[...]
        h1 = jnp.maximum(
            num / (s[:, None] + 1e-16)
            + jnp.dot(xb, ws1[...], preferred_element_type=F32) + bs1_r[...],
            0.0)
        h1_o[...] = h1
        q2_o[...] = (jnp.dot(h1, wq2[...], preferred_element_type=F32)
                     + bq2_r[...]) * scale2
        k2_o[...] = jnp.dot(h1, wk2[...], preferred_element_type=F32) + bk2_r[...]
        vlo_o[...] = jnp.dot(h1, wvlo[...], preferred_element_type=F32) + bvlo_r[...]
        vhi_o[...] = jnp.dot(h1, wvhi[...], preferred_element_type=F32) + bvhi_r[...]

    full = lambda a: pl.BlockSpec(a.shape, lambda i: (0,) * a.ndim)
    return pl.pallas_call(
        body,
        grid=(n // BN,),
        in_specs=[pl.BlockSpec((NC, BN, 16), lambda i: (0, i, 0)),
                  pl.BlockSpec((NC, BN), lambda i: (0, i)),
                  pl.BlockSpec((BN, f), lambda i: (i, 0)),
                  full(Ws1), full(bs1), full(Wq2), full(bq2),
                  full(Wk2), full(bk2), full(Wvlo), full(bvlo),
                  full(Wvhi), full(bvhi)],
        out_specs=[pl.BlockSpec((BN, 16), lambda i: (i, 0)),
                   pl.BlockSpec((BN, 32), lambda i: (i, 0)),
                   pl.BlockSpec((BN, 32), lambda i: (i, 0)),
                   pl.BlockSpec((BN, 16), lambda i: (i, 0)),
                   pl.BlockSpec((BN, 16), lambda i: (i, 0))],
        out_shape=[jax.ShapeDtypeStruct((n, 16), F32),
                   jax.ShapeDtypeStruct((n, 32), F32),
                   jax.ShapeDtypeStruct((n, 32), F32),
                   jax.ShapeDtypeStruct((n, 16), F32),
                   jax.ShapeDtypeStruct((n, 16), F32)],
    )(num1, s1, x_p, Ws1, bs1, Wq2, bq2, Wk2, bk2, Wvlo, bvlo, Wvhi, bvhi)


def _tc_finalize2_pool(numlo, numhi, s2, h1, Ws2, bs2, batch_p):
    n = h1.shape[0]

    def body(nlo_ref, nhi_ref, s_ref, h1_ref, ws2, bs2_r, b_ref, pooled_o):
        i = pl.program_id(0)
        num = jnp.concatenate(
            [nlo_ref[0] + nlo_ref[1], nhi_ref[0] + nhi_ref[1]], axis=1)
        s = s_ref[0] + s_ref[1]
        h2 = jnp.maximum(
            num / (s[:, None] + 1e-16)
            + jnp.dot(h1_ref[...], ws2[...], preferred_element_type=F32)
            + bs2_r[...],
            0.0)
        bb = b_ref[0]
        onehot = (lax.broadcasted_iota(jnp.int32, (NUM_G, BN), 0)
                  == bb).astype(F32)
        h2e = jnp.concatenate([h2, jnp.ones((BN, 1), F32)], axis=1)

        @pl.when(i == 0)
        def _():
            pooled_o[...] = jnp.zeros_like(pooled_o)

        pooled_o[...] += jnp.dot(onehot, h2e, preferred_element_type=F32)

    full = lambda a: pl.BlockSpec(a.shape, lambda i: (0,) * a.ndim)
    return pl.pallas_call(
        body,
        grid=(n // BN,),
        in_specs=[pl.BlockSpec((NC, BN, 16), lambda i: (0, i, 0)),
                  pl.BlockSpec((NC, BN, 16), lambda i: (0, i, 0)),
                  pl.BlockSpec((NC, BN), lambda i: (0, i)),
                  pl.BlockSpec((BN, 16), lambda i: (i, 0)),
                  full(Ws2), full(bs2),
                  pl.BlockSpec((1, 1, BN), lambda i: (i, 0, 0))],
        out_specs=pl.BlockSpec((NUM_G, 33), lambda i: (0, 0)),
        out_shape=jax.ShapeDtypeStruct((NUM_G, 33), F32),
    )(numlo, numhi, s2, h1, Ws2, bs2, batch_p)


def _tc_mlp(pooled, Wfc1, bfc1, Wfc2, bfc2):
    def body(p_ref, w1, b1, w2, b2, o_ref):
        p = p_ref[...]
        cnt = jnp.maximum(p[:, 32:33], 1.0)
        pm = p[:, :32] / cnt
        hh = jnp.maximum(
            jnp.dot(pm, w1[...], preferred_element_type=F32) + b1[...], 0.0)
        o_ref[...] = jnp.dot(hh, w2[...], preferred_element_type=F32) + b2[...]

    return pl.pallas_call(
        body,
        out_shape=jax.ShapeDtypeStruct((NUM_G, 1), F32),
    )(pooled, Wfc1, bfc1, Wfc2, bfc2)


def _sc_mesh():
    return plsc.VectorSubcoreMesh(core_axis_name="c", subcore_axis_name="s",
                                  num_cores=NC, num_subcores=NS)


def _zero_spmem(sid, zb, zs, num_sh, s_sh, d):
    npt = NP // NS
    z16 = jnp.zeros((16,), F32)

    @pl.loop(0, ZR)
    def _(i):
        for h in range(d // 16):
            zb[i, pl.ds(h * 16, 16)] = z16

    @pl.loop(0, ZR // 16)
    def _(i):
        zs[pl.ds(i * 16, 16)] = z16

    @pl.loop(0, npt // ZR)
    def _(i):
        base = sid * npt + i * ZR
        pltpu.sync_copy(zb, num_sh.at[pl.ds(base, ZR)])
        pltpu.sync_copy(zs, s_sh.at[pl.ds(base, ZR)])

    plsc.subcore_barrier()


def _sc_edge_l1(q1, k1, v1, srcg, dstg, ep):
    d = 16
    npt = NP // NS
    nchunk = ep // NW // CH

    @functools.partial(
        pl.kernel,
        out_type=(jax.ShapeDtypeStruct((NC, NP, d), F32),
                  jax.ShapeDtypeStruct((NC, NP), F32)),
        mesh=_sc_mesh(),
        compiler_params=pltpu.CompilerParams(needs_layout_passes=False, use_tc_tiling_on_sc=False),
        scratch_types=(
            pltpu.VMEM((GC, 128), jnp.int32),
            pltpu.VMEM((GC, 128), jnp.int32),
            pltpu.VMEM((CH, d), F32),
            pltpu.VMEM((CH, d), F32),
            pltpu.VMEM((CH, d), F32),
            pltpu.VMEM((CH, d), F32),
            pltpu.VMEM((CH,), F32),
            pltpu.VMEM((ZR, d), F32),
            pltpu.VMEM((ZR,), F32),
            pltpu.VMEM_SHARED((NP, d), F32),
            pltpu.VMEM_SHARED((NP,), F32),
            pltpu.SemaphoreType.DMA,
        ),
    )
    def k(q_hbm, k_hbm, v_hbm, srcg_hbm, dstg_hbm, num_out, s_out,
          srci, dsti, qb, kb, vb, evb, eb, zb, zs, num_sh, s_sh, sem):
        c = lax.axis_index("c")
        sid = lax.axis_index("s")
        _zero_spmem(sid, zb, zs, num_sh, s_sh, d)
        wid = c * NS + sid
        gbase = wid * (ep // NW // 128)

        @pl.loop(0, nchunk)
        def _(t):
            g0 = gbase + t * GC
            pltpu.sync_copy(srcg_hbm.at[pl.ds(g0, GC)], srci)
            pltpu.sync_copy(dstg_hbm.at[pl.ds(g0, GC)], dsti)
            cps = []
            for g in range(GC):
                r = pl.ds(g * 128, 128)
                cps.append(pltpu.async_copy(q_hbm.at[dsti.at[g]], qb.at[r], sem))
                cps.append(pltpu.async_copy(k_hbm.at[srci.at[g]], kb.at[r], sem))
                cps.append(pltpu.async_copy(v_hbm.at[srci.at[g]], vb.at[r], sem))
            for cp in cps:
                cp.wait()

            lane = jnp.arange(16, dtype=jnp.int32)

            @pl.loop(0, CH // 16)
            def _(gg):
                base = gg * 16
                lvec = jnp.zeros((16,), F32)
                for j in range(16):
                    e = base + j
                    prod = qb[e, :] * kb[e, :]
                    tot = jnp.full((16,), jnp.sum(prod), F32)
                    lvec = jnp.where(lane == j, tot, lvec)
                evv = jnp.exp(lvec)
                eb[pl.ds(base, 16)] = evv
                for j in range(16):
                    e = base + j
                    evb[e, :] = vb[e, :] * evv[j]

            for g in range(GC):
                r = pl.ds(g * 128, 128)
                pltpu.sync_copy(evb.at[r], num_sh.at[dsti.at[g]], add=True)
                pltpu.sync_copy(eb.at[r], s_sh.at[dsti.at[g]], add=True)

        plsc.subcore_barrier()
        fb = sid * npt
        pltpu.sync_copy(num_sh.at[pl.ds(fb, npt)],
                        num_out.at[c, pl.ds(fb, npt)])
        pltpu.sync_copy(s_sh.at[pl.ds(fb, npt)], s_out.at[c, pl.ds(fb, npt)])

    return k(q1, k1, v1, srcg, dstg)


def _sc_edge_l2_logits(q2, k2, srcg, dstg, ep):
    d = 32
    npt = NP // NS
    nchunk = ep // NW // CH

    @functools.partial(
        pl.kernel,
        out_type=(jax.ShapeDtypeStruct((NC, NP), F32),
                  jax.ShapeDtypeStruct((ep,), F32)),
        mesh=_sc_mesh(),
        compiler_params=pltpu.CompilerParams(needs_layout_passes=False, use_tc_tiling_on_sc=False),
        scratch_types=(
            pltpu.VMEM((GC, 128), jnp.int32),
            pltpu.VMEM((GC, 128), jnp.int32),
            pltpu.VMEM((CH, d), F32),
            pltpu.VMEM((CH, d), F32),
            pltpu.VMEM((CH,), F32),
            pltpu.VMEM((ZR,), F32),
            pltpu.VMEM_SHARED((NP,), F32),
            pltpu.SemaphoreType.DMA,
        ),
    )
    def k(q_hbm, k_hbm, srcg_hbm, dstg_hbm, s_out, e_out,
          srci, dsti, qb, kb, eb, zs, s_sh, sem):
        c = lax.axis_index("c")
        sid = lax.axis_index("s")
        z16 = jnp.zeros((16,), F32)

        @pl.loop(0, ZR // 16)
        def _(i):
            zs[pl.ds(i * 16, 16)] = z16

        @pl.loop(0, npt // ZR)
        def _(i):
            pltpu.sync_copy(zs, s_sh.at[pl.ds(sid * npt + i * ZR, ZR)])

        plsc.subcore_barrier()
        wid = c * NS + sid
        gbase = wid * (ep // NW // 128)

        @pl.loop(0, nchunk)
        def _(t):
            g0 = gbase + t * GC
            pltpu.sync_copy(srcg_hbm.at[pl.ds(g0, GC)], srci)
            pltpu.sync_copy(dstg_hbm.at[pl.ds(g0, GC)], dsti)
            cps = []
            for g in range(GC):
                r = pl.ds(g * 128, 128)
                cps.append(pltpu.async_copy(q_hbm.at[dsti.at[g]], qb.at[r], sem))
                cps.append(pltpu.async_copy(k_hbm.at[srci.at[g]], kb.at[r], sem))
            for cp in cps:
                cp.wait()

            lane = jnp.arange(16, dtype=jnp.int32)

            @pl.loop(0, CH // 16)
            def _(gg):
                base = gg * 16
                lvec = jnp.zeros((16,), F32)
                for j in range(16):
                    e = base + j
                    prod = (qb[e, pl.ds(0, 16)] * kb[e, pl.ds(0, 16)]
                            + qb[e, pl.ds(16, 16)] * kb[e, pl.ds(16, 16)])
                    tot = jnp.full((16,), jnp.sum(prod), F32)
                    lvec = jnp.where(lane == j, tot, lvec)
                eb[pl.ds(base, 16)] = jnp.exp(lvec)

            pltpu.sync_copy(eb, e_out.at[pl.ds(g0 * 128, CH)])
            for g in range(GC):
                r = pl.ds(g * 128, 128)
                pltpu.sync_copy(eb.at[r], s_sh.at[dsti.at[g]], add=True)

        plsc.subcore_barrier()
        fb = sid * npt
        pltpu.sync_copy(s_sh.at[pl.ds(fb, npt)], s_out.at[c, pl.ds(fb, npt)])

    return k(q2, k2, srcg, dstg)


def _sc_edge_l2_values(vh, e_all, srcg, dstg, ep):
    d = 16
    npt = NP // NS
    nchunk = ep // NW // CH

    @functools.partial(
        pl.kernel,
        out_type=jax.ShapeDtypeStruct((NC, NP, d), F32),
        mesh=_sc_mesh(),
        compiler_params=pltpu.CompilerParams(needs_layout_passes=False, use_tc_tiling_on_sc=False),
        scratch_types=(
            pltpu.VMEM((GC, 128), jnp.int32),
            pltpu.VMEM((GC, 128), jnp.int32),
            pltpu.VMEM((CH, d), F32),
            pltpu.VMEM((CH, d), F32),
            pltpu.VMEM((CH,), F32),
            pltpu.VMEM((ZR, d), F32),
            pltpu.VMEM_SHARED((NP, d), F32),
            pltpu.SemaphoreType.DMA,
        ),
    )
    def k(v_hbm, e_hbm, srcg_hbm, dstg_hbm, num_out,
          srci, dsti, vb, evb, eb, zb, num_sh, sem):
        c = lax.axis_index("c")
        sid = lax.axis_index("s")
        z16 = jnp.zeros((16,), F32)

        @pl.loop(0, ZR)
        def _(i):
            zb[i, :] = z16

        @pl.loop(0, npt // ZR)
        def _(i):
            pltpu.sync_copy(zb, num_sh.at[pl.ds(sid * npt + i * ZR, ZR)])

        plsc.subcore_barrier()
        wid = c * NS + sid
        gbase = wid * (ep // NW // 128)

        @pl.loop(0, nchunk)
        def _(t):
            g0 = gbase + t * GC
            pltpu.sync_copy(srcg_hbm.at[pl.ds(g0, GC)], srci)
            pltpu.sync_copy(dstg_hbm.at[pl.ds(g0, GC)], dsti)
            pltpu.sync_copy(e_hbm.at[pl.ds(g0 * 128, CH)], eb)
            cps = []
            for g in range(GC):
                r = pl.ds(g * 128, 128)
                cps.append(pltpu.async_copy(v_hbm.at[srci.at[g]], vb.at[r], sem))
            for cp in cps:
                cp.wait()

            @pl.loop(0, CH // 16)
            def _(gg):
                base = gg * 16
                evv = eb[pl.ds(base, 16)]
                for j in range(16):
                    e = base + j
                    evb[e, :] = vb[e, :] * evv[j]

            for g in range(GC):
                r = pl.ds(g * 128, 128)
                pltpu.sync_copy(evb.at[r], num_sh.at[dsti.at[g]], add=True)

        plsc.subcore_barrier()
        fb = sid * npt
        pltpu.sync_copy(num_sh.at[pl.ds(fb, npt)],
                        num_out.at[c, pl.ds(fb, npt)])

    return k(vh, e_all, srcg, dstg)


def kernel(x, edge_index, batch,
           Wq1, bq1, Wk1, bk1, Wv1, bv1, Ws1, bs1,
           Wq2, bq2, Wk2, bk2, Wv2, bv2, Ws2, bs2,
           Wfc1, bfc1, Wfc2, bfc2):
    n, f = x.shape
    e_cnt = edge_index.shape[1]
    ep = NW * CH * -(-e_cnt // (NW * CH))

    x_p = jnp.zeros((NP, f), F32).at[:n].set(x)
    src = edge_index[0]
    dst = edge_index[1]
    pad = ep - e_cnt
    srcg = jnp.concatenate(
        [src, jnp.zeros((pad,), src.dtype)]).reshape(ep // 128, 128)
    dstg = jnp.concatenate(
        [dst, jnp.full((pad,), n, dst.dtype)]).reshape(ep // 128, 128)
    batch_p = jnp.concatenate(
        [batch, jnp.full((NP - n,), NUM_G, batch.dtype)]).reshape(
            NP // BN, 1, BN)

    r = lambda b: b.reshape(1, -1)
    q1, k1, v1 = _tc_qkv(x_p, Wq1, r(bq1), Wk1, r(bk1), Wv1, r(bv1),
                         1.0 / (16.0 ** 0.5))
    num1, s1 = _sc_edge_l1(q1, k1, v1, srcg, dstg, ep)
    h1, q2, k2, v2lo, v2hi = _tc_finalize1_qkv2(
        num1, s1, x_p, Ws1, r(bs1), Wq2, r(bq2), Wk2, r(bk2),
        Wv2[:, :16], r(bv2[:16]), Wv2[:, 16:], r(bv2[16:]),
        1.0 / (32.0 ** 0.5))
    s2, e2 = _sc_edge_l2_logits(q2, k2, srcg, dstg, ep)
    num2lo = _sc_edge_l2_values(v2lo, e2, srcg, dstg, ep)
    num2hi = _sc_edge_l2_values(v2hi, e2, srcg, dstg, ep)
    pooled = _tc_finalize2_pool(num2lo, num2hi, s2, h1, Ws2, r(bs2), batch_p)
    return _tc_mlp(pooled, Wfc1, r(bfc1), Wfc2, r(bfc2))


# speedup vs baseline: 42.7896x; 1.1722x over previous
import functools

import jax
import jax.numpy as jnp
from jax import lax
from jax.experimental import pallas as pl
from jax.experimental.pallas import tpu as pltpu
from jax.experimental.pallas import tpu_sc as plsc

NC, NS = 2, 16
NW = NC * NS
NUM_G = 512
BN = 2048
ZR = 128
NP = 100352
EQ = NW * 512
F32 = jnp.float32


def _tc_qkv1(x_p, Wq, bq, Wk, bk, Wv, bv, scale):
    n, f = x_p.shape
    d = Wq.shape[1]

    def body(x_ref, wq, bq_r, wk, bk_r, wv, bv_r, q_o, kv_o):
        xb = x_ref[...]
        q_o[...] = (jnp.dot(xb, wq[...], preferred_element_type=F32)
                    + bq_r[...]) * scale
        kk = jnp.dot(xb, wk[...], preferred_element_type=F32) + bk_r[...]
        vv = jnp.dot(xb, wv[...], preferred_element_type=F32) + bv_r[...]
        kv_o[...] = jnp.concatenate([kk, vv], axis=1)

    full = lambda a: pl.BlockSpec(a.shape, lambda i: (0,) * a.ndim)
    return pl.pallas_call(
        body,
        grid=(n // BN,),
        in_specs=[pl.BlockSpec((BN, f), lambda i: (i, 0)),
                  full(Wq), full(bq), full(Wk), full(bk), full(Wv), full(bv)],
        out_specs=[pl.BlockSpec((BN, d), lambda i: (i, 0)),
                   pl.BlockSpec((BN, 2 * d), lambda i: (i, 0))],
        out_shape=[jax.ShapeDtypeStruct((n, d), F32),
                   jax.ShapeDtypeStruct((n, 2 * d), F32)],
    )(x_p, Wq, bq, Wk, bk, Wv, bv)


def _tc_finalize1_qkv2(num1, s1, x_p, Ws1, bs1, Wq2, bq2, Wk2, bk2,
                       Wvlo, bvlo, Wvhi, bvhi, scale2):
    n, f = x_p.shape

    def body(num_ref, s_ref, x_ref, ws1, bs1_r, wq2, bq2_r, wk2, bk2_r,
             wvlo, bvlo_r, wvhi, bvhi_r, h1_o, q2_o, k2_o, vlo_o, vhi_o):
        num = num_ref[0] + num_ref[1]
        s = s_ref[0] + s_ref[1]
        xb = x_ref[...]
        h1 = jnp.maximum(
            num / (s[:, None] + 1e-16)
            + jnp.dot(xb, ws1[...], preferred_element_type=F32) + bs1_r[...],
            0.0)
        h1_o[...] = h1
        q2_o[...] = (jnp.dot(h1, wq2[...], preferred_element_type=F32)
                     + bq2_r[...]) * scale2
        k2_o[...] = jnp.dot(h1, wk2[...], preferred_element_type=F32) + bk2_r[...]
        vlo_o[...] = jnp.dot(h1, wvlo[...], preferred_element_type=F32) + bvlo_r[...]
        vhi_o[...] = jnp.dot(h1, wvhi[...], preferred_element_type=F32) + bvhi_r[...]

    full = lambda a: pl.BlockSpec(a.shape, lambda i: (0,) * a.ndim)
    return pl.pallas_call(
        body,
        grid=(n // BN,),
        in_specs=[pl.BlockSpec((NC, BN, 16), lambda i: (0, i, 0)),
                  pl.BlockSpec((NC, BN), lambda i: (0, i)),
                  pl.BlockSpec((BN, f), lambda i: (i, 0)),
                  full(Ws1), full(bs1), full(Wq2), full(bq2),
                  full(Wk2), full(bk2), full(Wvlo), full(bvlo),
                  full(Wvhi), full(bvhi)],
        out_specs=[pl.BlockSpec((BN, 16), lambda i: (i, 0)),
                   pl.BlockSpec((BN, 32), lambda i: (i, 0)),
                   pl.BlockSpec((BN, 32), lambda i: (i, 0)),
                   pl.BlockSpec((BN, 16), lambda i: (i, 0)),
                   pl.BlockSpec((BN, 16), lambda i: (i, 0))],
        out_shape=[jax.ShapeDtypeStruct((n, 16), F32),
                   jax.ShapeDtypeStruct((n, 32), F32),
                   jax.ShapeDtypeStruct((n, 32), F32),
                   jax.ShapeDtypeStruct((n, 16), F32),
                   jax.ShapeDtypeStruct((n, 16), F32)],
    )(num1, s1, x_p, Ws1, bs1, Wq2, bq2, Wk2, bk2, Wvlo, bvlo, Wvhi, bvhi)


def _tc_finalize2_pool(numlo, numhi, s2, h1, Ws2, bs2, batch_p):
    n = h1.shape[0]

    def body(nlo_ref, nhi_ref, s_ref, h1_ref, ws2, bs2_r, b_ref, pooled_o):
        i = pl.program_id(0)
        num = jnp.concatenate(
            [nlo_ref[0] + nlo_ref[1], nhi_ref[0] + nhi_ref[1]], axis=1)
        s = s_ref[0] + s_ref[1]
        h2 = jnp.maximum(
            num / (s[:, None] + 1e-16)
            + jnp.dot(h1_ref[...], ws2[...], preferred_element_type=F32)
            + bs2_r[...],
            0.0)
        bb = b_ref[0]
        onehot = (lax.broadcasted_iota(jnp.int32, (NUM_G, BN), 0)
                  == bb).astype(F32)
        h2e = jnp.concatenate([h2, jnp.ones((BN, 1), F32)], axis=1)

        @pl.when(i == 0)
        def _():
            pooled_o[...] = jnp.zeros_like(pooled_o)

        pooled_o[...] += jnp.dot(onehot, h2e, preferred_element_type=F32)

    full = lambda a: pl.BlockSpec(a.shape, lambda i: (0,) * a.ndim)
    return pl.pallas_call(
        body,
        grid=(n // BN,),
        in_specs=[pl.BlockSpec((NC, BN, 16), lambda i: (0, i, 0)),
                  pl.BlockSpec((NC, BN, 16), lambda i: (0, i, 0)),
                  pl.BlockSpec((NC, BN), lambda i: (0, i)),
                  pl.BlockSpec((BN, 16), lambda i: (i, 0)),
                  full(Ws2), full(bs2),
                  pl.BlockSpec((1, 1, BN), lambda i: (i, 0, 0))],
        out_specs=pl.BlockSpec((NUM_G, 33), lambda i: (0, 0)),
        out_shape=jax.ShapeDtypeStruct((NUM_G, 33), F32),
    )(numlo, numhi, s2, h1, Ws2, bs2, batch_p)


def _tc_mlp(pooled, Wfc1, bfc1, Wfc2, bfc2):
    def body(p_ref, w1, b1, w2, b2, o_ref):
        p = p_ref[...]
        cnt = jnp.maximum(p[:, 32:33], 1.0)
        pm = p[:, :32] / cnt
        hh = jnp.maximum(
            jnp.dot(pm, w1[...], preferred_element_type=F32) + b1[...], 0.0)
        o_ref[...] = jnp.dot(hh, w2[...], preferred_element_type=F32) + b2[...]

    return pl.pallas_call(
        body,
        out_shape=jax.ShapeDtypeStruct((NUM_G, 1), F32),
    )(pooled, Wfc1, bfc1, Wfc2, bfc2)


_SC_PARAMS = pltpu.CompilerParams(needs_layout_passes=False,
                                  use_tc_tiling_on_sc=False)


def _sc_mesh():
    return plsc.VectorSubcoreMesh(core_axis_name="c", subcore_axis_name="s",
                                  num_cores=NC, num_subcores=NS)


def _logit_exp_group(qb, kb2, base, d, kcol0=0):
    lane = jnp.arange(16, dtype=jnp.int32)
    lvec = jnp.zeros((16,), F32)
    for j in range(16):
        e = base + j
        prod = qb[e, pl.ds(0, 16)] * kb2[e, pl.ds(kcol0, 16)]
        for h in range(1, d // 16):
            prod = prod + (qb[e, pl.ds(h * 16, 16)]
                           * kb2[e, pl.ds(kcol0 + h * 16, 16)])
        tot = jnp.full((16,), jnp.sum(prod), F32)
        lvec = jnp.where(lane == j, tot, lvec)
    return jnp.exp(lvec)


def _sc_edge_l1(q1, kv1, srcg, dstg, ep):
    d = 16
    ch, gc = 256, 2
    npt = NP // NS
    nchunk = ep // NW // ch

    @functools.partial(
        pl.kernel,
        out_type=(jax.ShapeDtypeStruct((NC, NP, d), F32),
                  jax.ShapeDtypeStruct((NC, NP), F32)),
        mesh=_sc_mesh(),
        compiler_params=_SC_PARAMS,
        scratch_types=(
            pltpu.VMEM((gc, 128), jnp.int32),
            pltpu.VMEM((gc, 128), jnp.int32),
            pltpu.VMEM((ch, d), F32),
            pltpu.VMEM((ch, 2 * d), F32),
            pltpu.VMEM((ch, d), F32),
            pltpu.VMEM((ch,), F32),
            pltpu.VMEM((ZR, d), F32),
            pltpu.VMEM((ZR,), F32),
            pltpu.VMEM_SHARED((NP, d), F32),
            pltpu.VMEM_SHARED((NP,), F32),
            pltpu.SemaphoreType.DMA,
        ),
    )
    def k(q_hbm, kv_hbm, srcg_hbm, dstg_hbm, num_out, s_out,
          srci, dsti, qb, kvb, evb, eb, zb, zs, num_sh, s_sh, sem):
        c = lax.axis_index("c")
        sid = lax.axis_index("s")
        _zero_spmem(sid, zb, zs, num_sh, s_sh, d)
        wid = c * NS + sid
        gbase = wid * (ep // NW // 128)

        @pl.loop(0, nchunk)
        def _(t):
            pltpu.sync_copy(srcg_hbm.at[pl.ds(gbase + t * gc, gc)], srci)
            pltpu.sync_copy(dstg_hbm.at[pl.ds(gbase + t * gc, gc)], dsti)
            cps = []
            for g in range(gc):
                r = pl.ds(g * 128, 128)
                cps.append(pltpu.async_copy(q_hbm.at[dsti.at[g]],
                                            qb.at[r], sem))
                cps.append(pltpu.async_copy(kv_hbm.at[srci.at[g]],
                                            kvb.at[r], sem))
            for cp in cps:
                cp.wait()

            @pl.loop(0, ch // 16)
            def _(gg):
                base = gg * 16
                evv = _logit_exp_group(qb, kvb, base, d, kcol0=0)
                eb[pl.ds(base, 16)] = evv
                for j in range(16):
                    e = base + j
                    evb[e, :] = kvb[e, pl.ds(d, 16)] * evv[j]

            for g in range(gc):
                r = pl.ds(g * 128, 128)
                pltpu.sync_copy(evb.at[r], num_sh.at[dsti.at[g]], add=True)
                pltpu.sync_copy(eb.at[r], s_sh.at[dsti.at[g]], add=True)

        plsc.subcore_barrier()
        fb = sid * npt
        pltpu.sync_copy(num_sh.at[pl.ds(fb, npt)],
                        num_out.at[c, pl.ds(fb, npt)])
        pltpu.sync_copy(s_sh.at[pl.ds(fb, npt)], s_out.at[c, pl.ds(fb, npt)])

    return k(q1, kv1, srcg, dstg)


def _sc_edge_l2_logits(q2, k2, srcg, dstg, ep):
    d = 32
    ch, gc = 512, 4
    npt = NP // NS
    nchunk = ep // NW // ch

    @functools.partial(
        pl.kernel,
        out_type=(jax.ShapeDtypeStruct((NC, NP), F32),
                  jax.ShapeDtypeStruct((ep,), F32)),
        mesh=_sc_mesh(),
        compiler_params=_SC_PARAMS,
        scratch_types=(
            pltpu.VMEM((gc, 128), jnp.int32),
            pltpu.VMEM((gc, 128), jnp.int32),
            pltpu.VMEM((ch, d), F32),
            pltpu.VMEM((ch, d), F32),
            pltpu.VMEM((ch,), F32),
            pltpu.VMEM((ZR,), F32),
            pltpu.VMEM_SHARED((NP,), F32),
            pltpu.SemaphoreType.DMA,
        ),
    )
    def k(q_hbm, k_hbm, srcg_hbm, dstg_hbm, s_out, e_out,
          srci, dsti, qb, kb, eb, zs, s_sh, sem):
        c = lax.axis_index("c")
        sid = lax.axis_index("s")
        z16 = jnp.zeros((16,), F32)

        @pl.loop(0, ZR // 16)
        def _(i):
            zs[pl.ds(i * 16, 16)] = z16

        @pl.loop(0, npt // ZR)
        def _(i):
            pltpu.sync_copy(zs, s_sh.at[pl.ds(sid * npt + i * ZR, ZR)])

        plsc.subcore_barrier()
        wid = c * NS + sid
        gbase = wid * (ep // NW // 128)

        @pl.loop(0, nchunk)
        def _(t):
            pltpu.sync_copy(srcg_hbm.at[pl.ds(gbase + t * gc, gc)], srci)
            pltpu.sync_copy(dstg_hbm.at[pl.ds(gbase + t * gc, gc)], dsti)
            cps = []
            for g in range(gc):
                r = pl.ds(g * 128, 128)
                cps.append(pltpu.async_copy(q_hbm.at[dsti.at[g]],
                                            qb.at[r], sem))
                cps.append(pltpu.async_copy(k_hbm.at[srci.at[g]],
                                            kb.at[r], sem))
            for cp in cps:
                cp.wait()

            @pl.loop(0, ch // 16)
            def _(gg):
                base = gg * 16
                eb[pl.ds(base, 16)] = _logit_exp_group(qb, kb, base, d)

            e0 = (gbase + t * gc) * 128
            pltpu.sync_copy(eb, e_out.at[pl.ds(e0, ch)])
            for g in range(gc):
                r = pl.ds(g * 128, 128)
                pltpu.sync_copy(eb.at[r], s_sh.at[dsti.at[g]], add=True)

        plsc.subcore_barrier()
        fb = sid * npt
        pltpu.sync_copy(s_sh.at[pl.ds(fb, npt)], s_out.at[c, pl.ds(fb, npt)])

    return k(q2, k2, srcg, dstg)


def _sc_edge_l2_values(vh, e_all, srcg, dstg, ep):
    d = 16
    ch, gc = 512, 4
    npt = NP // NS
    nchunk = ep // NW // ch

    @functools.partial(
        pl.kernel,
        out_type=jax.ShapeDtypeStruct((NC, NP, d), F32),
        mesh=_sc_mesh(),
        compiler_params=_SC_PARAMS,
        scratch_types=(
            pltpu.VMEM((gc, 128), jnp.int32),
            pltpu.VMEM((gc, 128), jnp.int32),
            pltpu.VMEM((ch, d), F32),
            pltpu.VMEM((ch, d), F32),
            pltpu.VMEM((ch,), F32),
            pltpu.VMEM((ZR, d), F32),
            pltpu.VMEM_SHARED((NP, d), F32),
            pltpu.SemaphoreType.DMA,
        ),
    )
    def k(v_hbm, e_hbm, srcg_hbm, dstg_hbm, num_out,
          srci, dsti, vb, evb, eb, zb, num_sh, sem):
        c = lax.axis_index("c")
        sid = lax.axis_index("s")
        z16 = jnp.zeros((16,), F32)

        @pl.loop(0, ZR)
        def _(i):
            zb[i, :] = z16

        @pl.loop(0, npt // ZR)
        def _(i):
            pltpu.sync_copy(zb, num_sh.at[pl.ds(sid * npt + i * ZR, ZR)])

        plsc.subcore_barrier()
        wid = c * NS + sid
        gbase = wid * (ep // NW // 128)

        @pl.loop(0, nchunk)
        def _(t):
            pltpu.sync_copy(srcg_hbm.at[pl.ds(gbase + t * gc, gc)], srci)
            pltpu.sync_copy(dstg_hbm.at[pl.ds(gbase + t * gc, gc)], dsti)
            e0 = (gbase + t * gc) * 128
            cps = [pltpu.async_copy(e_hbm.at[pl.ds(e0, ch)], eb, sem)]
            for g in range(gc):
                r = pl.ds(g * 128, 128)
                cps.append(pltpu.async_copy(v_hbm.at[srci.at[g]],
                                            vb.at[r], sem))
            for cp in cps:
                cp.wait()

            @pl.loop(0, ch // 16)
            def _(gg):
                base = gg * 16
                evv = eb[pl.ds(base, 16)]
                for j in range(16):
                    e = base + j
                    evb[e, :] = vb[e, :] * evv[j]

            for g in range(gc):
                r = pl.ds(g * 128, 128)
                pltpu.sync_copy(evb.at[r], num_sh.at[dsti.at[g]], add=True)

        plsc.subcore_barrier()
        fb = sid * npt
        pltpu.sync_copy(num_sh.at[pl.ds(fb, npt)],
                        num_out.at[c, pl.ds(fb, npt)])

    return k(vh, e_all, srcg, dstg)


def _zero_spmem(sid, zb, zs, num_sh, s_sh, d):
    npt = NP // NS
    z16 = jnp.zeros((16,), F32)

    @pl.loop(0, ZR)
    def _(i):
        for h in range(d // 16):
            zb[i, pl.ds(h * 16, 16)] = z16

    @pl.loop(0, ZR // 16)
    def _(i):
        zs[pl.ds(i * 16, 16)] = z16

    @pl.loop(0, npt // ZR)
    def _(i):
        base = sid * npt + i * ZR
        pltpu.sync_copy(zb, num_sh.at[pl.ds(base, ZR)])
        pltpu.sync_copy(zs, s_sh.at[pl.ds(base, ZR)])

    plsc.subcore_barrier()


def kernel(x, edge_index, batch,
           Wq1, bq1, Wk1, bk1, Wv1, bv1, Ws1, bs1,
           Wq2, bq2, Wk2, bk2, Wv2, bv2, Ws2, bs2,
           Wfc1, bfc1, Wfc2, bfc2):
    n, f = x.shape
    e_cnt = edge_index.shape[1]
    ep = EQ * -(-e_cnt // EQ)

    x_p = jnp.zeros((NP, f), F32).at[:n].set(x)
    src = edge_index[0]
    dst = edge_index[1]
    pad = ep - e_cnt
    srcp = jnp.concatenate(
        [src, jnp.zeros((pad,), src.dtype)]).reshape(ep // 128, 128)
    dstp = jnp.concatenate(
        [dst, jnp.full((pad,), n, dst.dtype)]).reshape(ep // 128, 128)
    batch_p = jnp.concatenate(
        [batch, jnp.full((NP - n,), NUM_G, batch.dtype)]).reshape(
            NP // BN, 1, BN)

    r = lambda b: b.reshape(1, -1)
    q1, kv1 = _tc_qkv1(x_p, Wq1, r(bq1), Wk1, r(bk1), Wv1, r(bv1),
                       1.0 / (16.0 ** 0.5))
    num1, s1 = _sc_edge_l1(q1, kv1, srcp, dstp, ep)
    h1, q2, k2, v2lo, v2hi = _tc_finalize1_qkv2(
        num1, s1, x_p, Ws1, r(bs1), Wq2, r(bq2), Wk2, r(bk2),
        Wv2[:, :16], r(bv2[:16]), Wv2[:, 16:], r(bv2[16:]),
        1.0 / (32.0 ** 0.5))
    s2, e2 = _sc_edge_l2_logits(q2, k2, srcp, dstp, ep)
    num2lo = _sc_edge_l2_values(v2lo, e2, srcp, dstp, ep)
    num2hi = _sc_edge_l2_values(v2hi, e2, srcp, dstp, ep)
    pooled = _tc_finalize2_pool(num2lo, num2hi, s2, h1, Ws2, r(bs2), batch_p)
    return _tc_mlp(pooled, Wfc1, r(bfc1), Wfc2, r(bfc2))


# speedup vs baseline: 50.9551x; 1.1908x over previous
import functools

import jax
import jax.numpy as jnp
from jax import lax
from jax.experimental import pallas as pl
from jax.experimental.pallas import tpu as pltpu
from jax.experimental.pallas import tpu_sc as plsc

NC, NS = 2, 16
NW = NC * NS
NUM_G = 512
BN = 2048
ZR = 128
NP = 100352
EQ = NW * 512
F32 = jnp.float32


def _tc_qkv1(x_p, Wq, bq, Wk, bk, Wv, bv, scale):
    n, f = x_p.shape
    d = Wq.shape[1]

    def body(x_ref, wq, bq_r, wk, bk_r, wv, bv_r, q_o, kv_o):
        xb = x_ref[...]
        q_o[...] = (jnp.dot(xb, wq[...], preferred_element_type=F32)
                    + bq_r[...]) * scale
        kk = jnp.dot(xb, wk[...], preferred_element_type=F32) + bk_r[...]
        vv = jnp.dot(xb, wv[...], preferred_element_type=F32) + bv_r[...]
        kv_o[...] = jnp.concatenate([kk, vv], axis=1)

    full = lambda a: pl.BlockSpec(a.shape, lambda i: (0,) * a.ndim)
    return pl.pallas_call(
        body,
        grid=(n // BN,),
        in_specs=[pl.BlockSpec((BN, f), lambda i: (i, 0)),
                  full(Wq), full(bq), full(Wk), full(bk), full(Wv), full(bv)],
        out_specs=[pl.BlockSpec((BN, d), lambda i: (i, 0)),
                   pl.BlockSpec((BN, 2 * d), lambda i: (i, 0))],
        out_shape=[jax.ShapeDtypeStruct((n, d), F32),
                   jax.ShapeDtypeStruct((n, 2 * d), F32)],
    )(x_p, Wq, bq, Wk, bk, Wv, bv)


def _tc_finalize1_qkv2(num1, s1, x_p, Ws1, bs1, Wq2, bq2, Wk2, bk2,
                       Wvlo, bvlo, Wvhi, bvhi, scale2):
    n, f = x_p.shape

    def body(num_ref, s_ref, x_ref, ws1, bs1_r, wq2, bq2_r, wk2, bk2_r,
             wvlo, bvlo_r, wvhi, bvhi_r, h1_o, q2_o, k2_o, vlo_o, vhi_o):
        num = num_ref[0] + num_ref[1]
        s = s_ref[0] + s_ref[1]
        xb = x_ref[...]
        h1 = jnp.maximum(
            num / (s[:, None] + 1e-16)
            + jnp.dot(xb, ws1[...], preferred_element_type=F32) + bs1_r[...],
            0.0)
        h1_o[...] = h1
        q2_o[...] = (jnp.dot(h1, wq2[...], preferred_element_type=F32)
                     + bq2_r[...]) * scale2
        k2_o[...] = jnp.dot(h1, wk2[...], preferred_element_type=F32) + bk2_r[...]
        vlo_o[...] = jnp.dot(h1, wvlo[...], preferred_element_type=F32) + bvlo_r[...]
        vhi_o[...] = jnp.dot(h1, wvhi[...], preferred_element_type=F32) + bvhi_r[...]

    full = lambda a: pl.BlockSpec(a.shape, lambda i: (0,) * a.ndim)
    return pl.pallas_call(
        body,
        grid=(n // BN,),
        in_specs=[pl.BlockSpec((NC, BN, 16), lambda i: (0, i, 0)),
                  pl.BlockSpec((NC, BN), lambda i: (0, i)),
                  pl.BlockSpec((BN, f), lambda i: (i, 0)),
                  full(Ws1), full(bs1), full(Wq2), full(bq2),
                  full(Wk2), full(bk2), full(Wvlo), full(bvlo),
                  full(Wvhi), full(bvhi)],
        out_specs=[pl.BlockSpec((BN, 16), lambda i: (i, 0)),
                   pl.BlockSpec((BN, 32), lambda i: (i, 0)),
                   pl.BlockSpec((BN, 32), lambda i: (i, 0)),
                   pl.BlockSpec((BN, 16), lambda i: (i, 0)),
                   pl.BlockSpec((BN, 16), lambda i: (i, 0))],
        out_shape=[jax.ShapeDtypeStruct((n, 16), F32),
                   jax.ShapeDtypeStruct((n, 32), F32),
                   jax.ShapeDtypeStruct((n, 32), F32),
                   jax.ShapeDtypeStruct((n, 16), F32),
                   jax.ShapeDtypeStruct((n, 16), F32)],
    )(num1, s1, x_p, Ws1, bs1, Wq2, bq2, Wk2, bk2, Wvlo, bvlo, Wvhi, bvhi)


def _tc_finalize2_pool(numlo, numhi, s2, h1, Ws2, bs2, batch_p):
    n = h1.shape[0]

    def body(nlo_ref, nhi_ref, s_ref, h1_ref, ws2, bs2_r, b_ref, pooled_o):
        i = pl.program_id(0)
        num = jnp.concatenate(
            [nlo_ref[0] + nlo_ref[1], nhi_ref[0] + nhi_ref[1]], axis=1)
        s = s_ref[0] + s_ref[1]
        h2 = jnp.maximum(
            num / (s[:, None] + 1e-16)
            + jnp.dot(h1_ref[...], ws2[...], preferred_element_type=F32)
            + bs2_r[...],
            0.0)
        bb = b_ref[0]
        onehot = (lax.broadcasted_iota(jnp.int32, (NUM_G, BN), 0)
                  == bb).astype(F32)
        h2e = jnp.concatenate([h2, jnp.ones((BN, 1), F32)], axis=1)

        @pl.when(i == 0)
        def _():
            pooled_o[...] = jnp.zeros_like(pooled_o)

        pooled_o[...] += jnp.dot(onehot, h2e, preferred_element_type=F32)

    full = lambda a: pl.BlockSpec(a.shape, lambda i: (0,) * a.ndim)
    return pl.pallas_call(
        body,
        grid=(n // BN,),
        in_specs=[pl.BlockSpec((NC, BN, 16), lambda i: (0, i, 0)),
                  pl.BlockSpec((NC, BN, 16), lambda i: (0, i, 0)),
                  pl.BlockSpec((NC, BN), lambda i: (0, i)),
                  pl.BlockSpec((BN, 16), lambda i: (i, 0)),
                  full(Ws2), full(bs2),
                  pl.BlockSpec((1, 1, BN), lambda i: (i, 0, 0))],
        out_specs=pl.BlockSpec((NUM_G, 33), lambda i: (0, 0)),
        out_shape=jax.ShapeDtypeStruct((NUM_G, 33), F32),
    )(numlo, numhi, s2, h1, Ws2, bs2, batch_p)


def _tc_mlp(pooled, Wfc1, bfc1, Wfc2, bfc2):
    def body(p_ref, w1, b1, w2, b2, o_ref):
        p = p_ref[...]
        cnt = jnp.maximum(p[:, 32:33], 1.0)
        pm = p[:, :32] / cnt
        hh = jnp.maximum(
            jnp.dot(pm, w1[...], preferred_element_type=F32) + b1[...], 0.0)
        o_ref[...] = jnp.dot(hh, w2[...], preferred_element_type=F32) + b2[...]

    return pl.pallas_call(
        body,
        out_shape=jax.ShapeDtypeStruct((NUM_G, 1), F32),
    )(pooled, Wfc1, bfc1, Wfc2, bfc2)


_SC_PARAMS = pltpu.CompilerParams(needs_layout_passes=False,
                                  use_tc_tiling_on_sc=False)


def _sc_mesh():
    return plsc.VectorSubcoreMesh(core_axis_name="c", subcore_axis_name="s",
                                  num_cores=NC, num_subcores=NS)


def _logit_exp_group(qb, kb2, base, d, kcol0=0):
    lane = jnp.arange(16, dtype=jnp.int32)
    lvec = jnp.zeros((16,), F32)
    for j in range(16):
        e = base + j
        prod = qb[e, pl.ds(0, 16)] * kb2[e, pl.ds(kcol0, 16)]
        for h in range(1, d // 16):
            prod = prod + (qb[e, pl.ds(h * 16, 16)]
                           * kb2[e, pl.ds(kcol0 + h * 16, 16)])
        tot = jnp.full((16,), jnp.sum(prod), F32)
        lvec = jnp.where(lane == j, tot, lvec)
    return jnp.exp(lvec)


def _sc_edge_l1(q1, kv1, sdil, ep):
    d = 16
    ch, gc = 128, 1
    npt = NP // NS
    nchunk = ep // NW // ch

    @functools.partial(
        pl.kernel,
        out_type=(jax.ShapeDtypeStruct((NC, NP, d), F32),
                  jax.ShapeDtypeStruct((NC, NP), F32)),
        mesh=_sc_mesh(),
        compiler_params=_SC_PARAMS,
        scratch_types=(
            pltpu.VMEM((4 * gc, 128), jnp.int32),
            pltpu.VMEM((ch, d), F32),
            pltpu.VMEM((ch, d), F32),
            pltpu.VMEM((ch, 2 * d), F32),
            pltpu.VMEM((ch, 2 * d), F32),
            pltpu.VMEM((ch, d), F32),
            pltpu.VMEM((ch,), F32),
            pltpu.VMEM((ZR, d), F32),
            pltpu.VMEM((ZR,), F32),
            pltpu.VMEM_SHARED((NP, d), F32),
            pltpu.VMEM_SHARED((NP,), F32),
            pltpu.SemaphoreType.DMA,
        ),
    )
    def k(q_hbm, kv_hbm, sdil_hbm, num_out, s_out,
          islab, qba, qbb, kvba, kvbb, evb, eb, zb, zs, num_sh, s_sh, sem):
        c = lax.axis_index("c")
        sid = lax.axis_index("s")
        _zero_spmem(sid, zb, zs, num_sh, s_sh, d)
        wid = c * NS + sid
        gbase = wid * (ep // NW // 128)
        qbs, kvbs = (qba, qbb), (kvba, kvbb)

        @pl.loop(0, nchunk // 2)
        def _(u):
            pltpu.sync_copy(
                sdil_hbm.at[pl.ds(2 * (gbase + 2 * u * gc), 4 * gc)], islab)
            dsc = []
            for h in range(2):
                cps = []
                for g in range(gc):
                    r = pl.ds(g * 128, 128)
                    srow, drow = 2 * (h * gc + g), 2 * (h * gc + g) + 1
                    cps.append(pltpu.async_copy(
                        q_hbm.at[islab.at[drow]], qbs[h].at[r], sem))
                    cps.append(pltpu.async_copy(
                        kv_hbm.at[islab.at[srow]], kvbs[h].at[r], sem))
                dsc.append(cps)
            for h in range(2):
                for cp in dsc[h]:
                    cp.wait()
                qb, kvb = qbs[h], kvbs[h]

                @pl.loop(0, ch // 16)
                def _(gg):
                    base = gg * 16
                    evv = _logit_exp_group(qb, kvb, base, d, kcol0=0)
                    eb[pl.ds(base, 16)] = evv
                    for j in range(16):
                        e = base + j
                        evb[e, :] = kvb[e, pl.ds(d, 16)] * evv[j]

                for g in range(gc):
                    r = pl.ds(g * 128, 128)
                    drow = 2 * (h * gc + g) + 1
                    pltpu.sync_copy(evb.at[r], num_sh.at[islab.at[drow]],
                                    add=True)
                    pltpu.sync_copy(eb.at[r], s_sh.at[islab.at[drow]],
                                    add=True)

        plsc.subcore_barrier()
        fb = sid * npt
        pltpu.sync_copy(num_sh.at[pl.ds(fb, npt)],
                        num_out.at[c, pl.ds(fb, npt)])
        pltpu.sync_copy(s_sh.at[pl.ds(fb, npt)], s_out.at[c, pl.ds(fb, npt)])

    return k(q1, kv1, sdil)


def _sc_edge_l2_logits(q2, k2, sdil, ep):
    d = 32
    ch, gc = 512, 4
    npt = NP // NS
    nchunk = ep // NW // ch

    @functools.partial(
        pl.kernel,
        out_type=(jax.ShapeDtypeStruct((NC, NP), F32),
                  jax.ShapeDtypeStruct((ep,), F32)),
        mesh=_sc_mesh(),
        compiler_params=_SC_PARAMS,
        scratch_types=(
            pltpu.VMEM((4 * gc, 128), jnp.int32),
            pltpu.VMEM((ch, d), F32),
            pltpu.VMEM((ch, d), F32),
            pltpu.VMEM((ch, d), F32),
            pltpu.VMEM((ch, d), F32),
            pltpu.VMEM((ch,), F32),
            pltpu.VMEM((ZR,), F32),
            pltpu.VMEM_SHARED((NP,), F32),
            pltpu.SemaphoreType.DMA,
        ),
    )
    def k(q_hbm, k_hbm, sdil_hbm, s_out, e_out,
          islab, qba, qbb, kba, kbb, eb, zs, s_sh, sem):
        c = lax.axis_index("c")
        sid = lax.axis_index("s")
        z16 = jnp.zeros((16,), F32)

        @pl.loop(0, ZR // 16)
        def _(i):
            zs[pl.ds(i * 16, 16)] = z16

        @pl.loop(0, npt // ZR)
        def _(i):
            pltpu.sync_copy(zs, s_sh.at[pl.ds(sid * npt + i * ZR, ZR)])

        plsc.subcore_barrier()
        wid = c * NS + sid
        gbase = wid * (ep // NW // 128)
        qbs, kbs = (qba, qbb), (kba, kbb)

        @pl.loop(0, nchunk // 2)
        def _(u):
            pltpu.sync_copy(
                sdil_hbm.at[pl.ds(2 * (gbase + 2 * u * gc), 4 * gc)], islab)
            dsc = []
            for h in range(2):
                cps = []
                for g in range(gc):
                    r = pl.ds(g * 128, 128)
                    srow, drow = 2 * (h * gc + g), 2 * (h * gc + g) + 1
                    cps.append(pltpu.async_copy(
                        q_hbm.at[islab.at[drow]], qbs[h].at[r], sem))
                    cps.append(pltpu.async_copy(
                        k_hbm.at[islab.at[srow]], kbs[h].at[r], sem))
                dsc.append(cps)
            for h in range(2):
                for cp in dsc[h]:
                    cp.wait()
                qb, kb = qbs[h], kbs[h]

                @pl.loop(0, ch // 16)
                def _(gg):
                    base = gg * 16
                    eb[pl.ds(base, 16)] = _logit_exp_group(qb, kb, base, d)

                e0 = (gbase + (2 * u + h) * gc) * 128
                pltpu.sync_copy(eb, e_out.at[pl.ds(e0, ch)])
                for g in range(gc):
                    r = pl.ds(g * 128, 128)
                    drow = 2 * (h * gc + g) + 1
                    pltpu.sync_copy(eb.at[r], s_sh.at[islab.at[drow]],
                                    add=True)

        plsc.subcore_barrier()
        fb = sid * npt
        pltpu.sync_copy(s_sh.at[pl.ds(fb, npt)], s_out.at[c, pl.ds(fb, npt)])

    return k(q2, k2, sdil)


def _sc_edge_l2_values(vh, e_all, sdil, ep):
    d = 16
    ch, gc = 256, 2
    npt = NP // NS
    nchunk = ep // NW // ch

    @functools.partial(
        pl.kernel,
        out_type=jax.ShapeDtypeStruct((NC, NP, d), F32),
        mesh=_sc_mesh(),
        compiler_params=_SC_PARAMS,
        scratch_types=(
            pltpu.VMEM((4 * gc, 128), jnp.int32),
            pltpu.VMEM((ch, d), F32),
            pltpu.VMEM((ch, d), F32),
            pltpu.VMEM((ch,), F32),
            pltpu.VMEM((ch,), F32),
            pltpu.VMEM((ch, d), F32),
            pltpu.VMEM((ZR, d), F32),
            pltpu.VMEM_SHARED((NP, d), F32),
            pltpu.SemaphoreType.DMA,
        ),
    )
    def k(v_hbm, e_hbm, sdil_hbm, num_out,
          islab, vba, vbb, eba, ebb, evb, zb, num_sh, sem):
        c = lax.axis_index("c")
        sid = lax.axis_index("s")
        z16 = jnp.zeros((16,), F32)

        @pl.loop(0, ZR)
        def _(i):
            zb[i, :] = z16

        @pl.loop(0, npt // ZR)
        def _(i):
            pltpu.sync_copy(zb, num_sh.at[pl.ds(sid * npt + i * ZR, ZR)])

        plsc.subcore_barrier()
        wid = c * NS + sid
        gbase = wid * (ep // NW // 128)
        vbs, ebs = (vba, vbb), (eba, ebb)

        @pl.loop(0, nchunk // 2)
        def _(u):
            pltpu.sync_copy(
                sdil_hbm.at[pl.ds(2 * (gbase + 2 * u * gc), 4 * gc)], islab)
            dsc = []
            for h in range(2):
                cps = []
                e0 = (gbase + (2 * u + h) * gc) * 128
                cps.append(pltpu.async_copy(e_hbm.at[pl.ds(e0, ch)],
                                            ebs[h], sem))
                for g in range(gc):
                    r = pl.ds(g * 128, 128)
                    srow = 2 * (h * gc + g)
                    cps.append(pltpu.async_copy(
                        v_hbm.at[islab.at[srow]], vbs[h].at[r], sem))
                dsc.append(cps)
            for h in range(2):
                for cp in dsc[h]:
                    cp.wait()
                vb, eb = vbs[h], ebs[h]

                @pl.loop(0, ch // 16)
                def _(gg):
                    base = gg * 16
                    evv = eb[pl.ds(base, 16)]
                    for j in range(16):
                        e = base + j
                        evb[e, :] = vb[e, :] * evv[j]

                for g in range(gc):
                    r = pl.ds(g * 128, 128)
                    drow = 2 * (h * gc + g) + 1
                    pltpu.sync_copy(evb.at[r], num_sh.at[islab.at[drow]],
                                    add=True)

        plsc.subcore_barrier()
        fb = sid * npt
        pltpu.sync_copy(num_sh.at[pl.ds(fb, npt)],
                        num_out.at[c, pl.ds(fb, npt)])

    return k(vh, e_all, sdil)


def _zero_spmem(sid, zb, zs, num_sh, s_sh, d):
    npt = NP // NS
    z16 = jnp.zeros((16,), F32)

    @pl.loop(0, ZR)
    def _(i):
        for h in range(d // 16):
            zb[i, pl.ds(h * 16, 16)] = z16

    @pl.loop(0, ZR // 16)
    def _(i):
        zs[pl.ds(i * 16, 16)] = z16

    @pl.loop(0, npt // ZR)
    def _(i):
        base = sid * npt + i * ZR
        pltpu.sync_copy(zb, num_sh.at[pl.ds(base, ZR)])
        pltpu.sync_copy(zs, s_sh.at[pl.ds(base, ZR)])

    plsc.subcore_barrier()


def kernel(x, edge_index, batch,
           Wq1, bq1, Wk1, bk1, Wv1, bv1, Ws1, bs1,
           Wq2, bq2, Wk2, bk2, Wv2, bv2, Ws2, bs2,
           Wfc1, bfc1, Wfc2, bfc2):
    n, f = x.shape
    e_cnt = edge_index.shape[1]
    ep = EQ * -(-e_cnt // EQ)

    x_p = jnp.zeros((NP, f), F32).at[:n].set(x)
    src = edge_index[0]
    dst = edge_index[1]
    pad = ep - e_cnt
    srcp = jnp.concatenate(
        [src, jnp.zeros((pad,), src.dtype)]).reshape(ep // 128, 128)
    dstp = jnp.concatenate(
        [dst, jnp.full((pad,), n, dst.dtype)]).reshape(ep // 128, 128)
    sdil = jnp.stack([srcp, dstp], axis=1).reshape(2 * (ep // 128), 128)
    batch_p = jnp.concatenate(
        [batch, jnp.full((NP - n,), NUM_G, batch.dtype)]).reshape(
            NP // BN, 1, BN)

    r = lambda b: b.reshape(1, -1)
    q1, kv1 = _tc_qkv1(x_p, Wq1, r(bq1), Wk1, r(bk1), Wv1, r(bv1),
                       1.0 / (16.0 ** 0.5))
    num1, s1 = _sc_edge_l1(q1, kv1, sdil, ep)
    h1, q2, k2, v2lo, v2hi = _tc_finalize1_qkv2(
        num1, s1, x_p, Ws1, r(bs1), Wq2, r(bq2), Wk2, r(bk2),
        Wv2[:, :16], r(bv2[:16]), Wv2[:, 16:], r(bv2[16:]),
        1.0 / (32.0 ** 0.5))
    s2, e2 = _sc_edge_l2_logits(q2, k2, sdil, ep)
    num2lo = _sc_edge_l2_values(v2lo, e2, sdil, ep)
    num2hi = _sc_edge_l2_values(v2hi, e2, sdil, ep)
    pooled = _tc_finalize2_pool(num2lo, num2hi, s2, h1, Ws2, r(bs2), batch_p)
    return _tc_mlp(pooled, Wfc1, r(bfc1), Wfc2, r(bfc2))
